# Initial kernel scaffold; baseline (speedup 1.0000x reference)
import jax
import jax.numpy as jnp
from jax.experimental import pallas as pl


def kernel(sentence1, sentence1_len_old, sentence2, sentence2_len_old, emb_table, W_sel, b_sel, is_train):
    raise NotImplementedError("write your pallas kernel here")


# speedup vs baseline: 12.6723x; 12.6723x over previous
import functools

import jax
import jax.numpy as jnp
from jax import lax
from jax.experimental import pallas as pl
from jax.experimental.pallas import tpu as pltpu
from jax.experimental.pallas import tpu_sc as plsc

_B, _L, _V, _D = 16, 4096, 1000000, 128
_R = 2 * _B
_NT = _R * _L
_NC, _NS = 2, 16
_NW = _NC * _NS
_TPW = _NT // _NW
_CH = 128

def _wid():
    return lax.axis_index("s") * _NC + lax.axis_index("c")


@functools.cache
def _make_sc_gather():
    mesh = plsc.VectorSubcoreMesh(core_axis_name="c", subcore_axis_name="s")

    @functools.partial(
        pl.kernel,
        mesh=mesh,
        compiler_params=pltpu.CompilerParams(needs_layout_passes=False),
        out_type=jax.ShapeDtypeStruct((_NT, _D), jnp.float32),
        scratch_types=[
            pltpu.VMEM((_CH,), jnp.int32),
            pltpu.VMEM((_CH, _D), jnp.float32),
            pltpu.SemaphoreType.DMA,
        ],
    )
    def _sc_gather(tok_hbm, emb_hbm, out_hbm, idx_v, rows_v, sem):
        base = _wid() * _TPW

        def chunk(i, carry):
            start = base + i * _CH
            pltpu.sync_copy(tok_hbm.at[pl.ds(start, _CH)], idx_v)
            pltpu.async_copy(emb_hbm.at[idx_v], rows_v, sem).wait()
            pltpu.sync_copy(rows_v, out_hbm.at[pl.ds(start, _CH)])
            return carry

        lax.fori_loop(0, _TPW // _CH, chunk, 0)

    return _sc_gather


_BLK = 8192


def _matvec_body(rows_ref, w_ref, b_ref, out_ref):
    out_ref[...] = (
        jnp.dot(rows_ref[...], w_ref[...], preferred_element_type=jnp.float32)
        + b_ref[0, 0]
    )


_tc_matvec = pl.pallas_call(
    _matvec_body,
    grid=(_NT // _BLK,),
    in_specs=[
        pl.BlockSpec((_BLK, _D), lambda i: (i, 0)),
        pl.BlockSpec((_D, 1), lambda i: (0, 0)),
        pl.BlockSpec((1, 1), lambda i: (0, 0)),
    ],
    out_specs=pl.BlockSpec((_BLK, 1), lambda i: (i, 0)),
    out_shape=jax.ShapeDtypeStruct((_NT, 1), jnp.float32),
)


def _post_body(tok_ref, sc_ref, u_ref, sel_ref, len_ref, logp_ref, zs_ref, zd_ref):
    eps = 1e-8
    p = jax.nn.sigmoid(sc_ref[...])
    sel = (u_ref[...] < p).astype(jnp.int32)
    sel_ref[...] = sel
    len_ref[...] = jnp.sum(sel, axis=1, keepdims=True)
    nz = tok_ref[...] != 0
    mf = nz.astype(jnp.float32)
    self_f = sel.astype(jnp.float32)
    logp = self_f * jnp.log(p + eps) + (1.0 - self_f) * jnp.log(1.0 - p + eps)
    logp_ref[...] = jnp.sum(logp * mf, axis=1, keepdims=True)
    ms = sel * nz.astype(jnp.int32)
    zs_ref[...] = jnp.sum(ms, axis=1, keepdims=True).astype(jnp.float32)
    d = jnp.abs(ms[:, 1:] - ms[:, :-1])
    zd_ref[...] = jnp.sum(d, axis=1, keepdims=True).astype(jnp.float32)


_tc_post = pl.pallas_call(
    _post_body,
    out_shape=(
        jax.ShapeDtypeStruct((_R, _L), jnp.int32),
        jax.ShapeDtypeStruct((_R, 1), jnp.int32),
        jax.ShapeDtypeStruct((_R, 1), jnp.float32),
        jax.ShapeDtypeStruct((_R, 1), jnp.float32),
        jax.ShapeDtypeStruct((_R, 1), jnp.float32),
    ),
)


@functools.cache
def _make_sc_compact():
    mesh = plsc.VectorSubcoreMesh(core_axis_name="c", subcore_axis_name="s")

    @functools.partial(
        pl.kernel,
        mesh=mesh,
        compiler_params=pltpu.CompilerParams(needs_layout_passes=False),
        out_type=jax.ShapeDtypeStruct((_R, _L), jnp.int32),
        scratch_types=[
            pltpu.VMEM((_L,), jnp.int32),
            pltpu.VMEM((_L,), jnp.int32),
            pltpu.VMEM((_L,), jnp.int32),
        ],
    )
    def _sc_compact(tok_hbm, sel_hbm, out_hbm, tok_v, sel_v, out_v):
        row = _wid()
        pltpu.sync_copy(tok_hbm.at[row], tok_v)
        pltpu.sync_copy(sel_hbm.at[row], sel_v)

        zero = jnp.zeros((16,), jnp.int32)

        def zbody(i, carry):
            out_v[pl.ds(i * 16, 16)] = zero
            return carry

        lax.fori_loop(0, _L // 16, zbody, 0)

        def body(i, off):
            s = sel_v[pl.ds(i * 16, 16)]
            t = tok_v[pl.ds(i * 16, 16)]
            pos = plsc.cumsum(s) - 1 + off
            plsc.store_scatter(out_v, [pos], t, mask=s > 0)
            return off + jnp.sum(s)

        lax.fori_loop(0, _L // 16, body, jnp.int32(0))
        pltpu.sync_copy(out_v, out_hbm.at[row])

    return _sc_compact


def kernel(sentence1, sentence1_len_old, sentence2, sentence2_len_old,
           emb_table, W_sel, b_sel, is_train):
    key = jax.random.key(42)
    k1, k2 = jax.random.split(key)
    u1 = jax.random.uniform(k1, (_B, _L))
    u2 = jax.random.uniform(k2, (_B, _L))
    u = jnp.concatenate([u1, u2], axis=0)
    tok = jnp.concatenate([sentence1, sentence2], axis=0)

    rows = _make_sc_gather()(tok.reshape(_NT), emb_table)
    scores = _tc_matvec(rows, W_sel, b_sel.reshape(1, 1))
    sel, lens, logp, zs, zd = _tc_post(tok, scores.reshape(_R, _L), u)
    selected = _make_sc_compact()(tok, sel)

    logpz = logp[:_B, 0] + logp[_B:, 0]
    zsum = zs[:_B, 0] + zs[_B:, 0]
    zdiff = zd[:_B, 0] + zd[_B:, 0]
    flag = is_train == 1
    logpz = jnp.where(flag, logpz, -1.0)
    zsum = jnp.where(flag, zsum, -1.0)
    zdiff = jnp.where(flag, zdiff, -1.0)
    return (selected[:_B], lens[:_B, 0], selected[_B:], lens[_B:, 0],
            logpz, zsum, zdiff)
